# baseline (speedup 1.0000x reference)
import functools

import jax
import jax.numpy as jnp
from jax import lax
from jax.experimental import pallas as pl
from jax.experimental.pallas import tpu as pltpu
from jax.experimental.pallas import tpu_sc as plsc

RANK = 32
NMODE = 3
LANES = 16

_info = plsc.get_sparse_core_info()
_NC, _NS = _info.num_cores, _info.num_subcores
_NW = _NC * _NS


def _make_kernel(batch: int):
    bpw = batch // _NW

    mesh = plsc.VectorSubcoreMesh(core_axis_name="c", subcore_axis_name="s")

    @functools.partial(
        pl.kernel,
        mesh=mesh,
        out_type=jax.ShapeDtypeStruct((batch,), jnp.float32),
        compiler_params=pltpu.CompilerParams(
            needs_layout_passes=False, use_tc_tiling_on_sc=False),
        scratch_types=[
            pltpu.VMEM((bpw * NMODE,), jnp.int32),
            pltpu.VMEM((bpw,), jnp.int32),
            pltpu.VMEM((bpw,), jnp.int32),
            pltpu.VMEM((bpw,), jnp.int32),
            pltpu.VMEM((bpw, RANK), jnp.float32),
            pltpu.VMEM((bpw, RANK), jnp.float32),
            pltpu.VMEM((bpw, RANK), jnp.float32),
            pltpu.VMEM((bpw,), jnp.float32),
            pltpu.SemaphoreType.DMA,
            pltpu.SemaphoreType.DMA,
            pltpu.SemaphoreType.DMA,
        ],
    )
    def cpd_kernel(idx_hbm, e0_hbm, e1_hbm, e2_hbm, out_hbm,
                   iflat_v, i0_v, i1_v, i2_v, r0_v, r1_v, r2_v, out_v,
                   sem0, sem1, sem2):
        wid = lax.axis_index("s") * _NC + lax.axis_index("c")
        base = wid * bpw

        pltpu.sync_copy(idx_hbm.at[pl.ds(base * NMODE, bpw * NMODE)], iflat_v)

        lane = lax.iota(jnp.int32, LANES)

        def deint_body(g, carry):
            flat0 = (g * LANES + lane) * NMODE
            i0_v[pl.ds(g * LANES, LANES)] = plsc.load_gather(iflat_v, [flat0])
            i1_v[pl.ds(g * LANES, LANES)] = plsc.load_gather(iflat_v, [flat0 + 1])
            i2_v[pl.ds(g * LANES, LANES)] = plsc.load_gather(iflat_v, [flat0 + 2])
            return carry

        lax.fori_loop(0, bpw // LANES, deint_body, 0)

        cp0 = pltpu.async_copy(e0_hbm.at[i0_v], r0_v, sem0)
        cp1 = pltpu.async_copy(e1_hbm.at[i1_v], r1_v, sem1)
        cp2 = pltpu.async_copy(e2_hbm.at[i2_v], r2_v, sem2)
        cp0.wait()
        cp1.wait()
        cp2.wait()

        def group_body(g, carry):
            b0 = g * LANES
            acc = jnp.zeros((LANES,), jnp.float32)
            for j in range(LANES):
                b = b0 + j
                lo = (r0_v[b, pl.ds(0, LANES)] * r1_v[b, pl.ds(0, LANES)]
                      * r2_v[b, pl.ds(0, LANES)])
                hi = (r0_v[b, pl.ds(LANES, LANES)] * r1_v[b, pl.ds(LANES, LANES)]
                      * r2_v[b, pl.ds(LANES, LANES)])
                acc = jnp.where(lane == j, jnp.sum(lo + hi), acc)
            out_v[pl.ds(b0, LANES)] = acc
            return carry

        lax.fori_loop(0, bpw // LANES, group_body, 0)

        pltpu.sync_copy(out_v, out_hbm.at[pl.ds(base, bpw)])

    return cpd_kernel


def kernel(idxs, E0, E1, E2):
    batch = idxs.shape[0]
    idx_flat = idxs.astype(jnp.int32).reshape(batch * NMODE)
    return _make_kernel(batch)(idx_flat, E0, E1, E2)
